# baseline (speedup 1.0000x reference)
import functools

import jax
import jax.numpy as jnp
from jax import lax
from jax.experimental import pallas as pl
from jax.experimental.pallas import tpu as pltpu
from jax.experimental.pallas import tpu_sc as plsc

_VOCAB = 1000000
_FEATURES = 32
_BATCH = 16384

_info = plsc.get_sparse_core_info()
_NC, _NS = _info.num_cores, _info.num_subcores
_NW = _NC * _NS
_B_PER_W = _BATCH // _NW

_mesh = plsc.VectorSubcoreMesh(core_axis_name="c", subcore_axis_name="s")


@functools.partial(
    pl.kernel,
    mesh=_mesh,
    compiler_params=pltpu.CompilerParams(use_tc_tiling_on_sc=False),
    out_type=(
        jax.ShapeDtypeStruct((_BATCH, _FEATURES), jnp.float32),
        jax.ShapeDtypeStruct((_BATCH, _FEATURES), jnp.float32),
    ),
    scratch_types=[
        pltpu.VMEM((_B_PER_W,), jnp.int32),
        pltpu.VMEM((_B_PER_W, _FEATURES), jnp.float32),
        pltpu.VMEM((_B_PER_W, _FEATURES), jnp.float32),
        pltpu.SemaphoreType.DMA,
        pltpu.SemaphoreType.DMA,
    ],
)
def _dual_gather(real_hbm, imag_hbm, x_hbm, out_r_hbm, out_i_hbm,
                 idx_v, rows_r, rows_i, sem_r, sem_i):
    wid = lax.axis_index("s") * _NC + lax.axis_index("c")
    base = wid * _B_PER_W
    pltpu.sync_copy(x_hbm.at[pl.ds(base, _B_PER_W)], idx_v)
    cp_r = pltpu.async_copy(real_hbm.at[idx_v], rows_r, sem_r)
    cp_i = pltpu.async_copy(imag_hbm.at[idx_v], rows_i, sem_i)
    cp_r.wait()
    pltpu.sync_copy(rows_r, out_r_hbm.at[pl.ds(base, _B_PER_W)])
    cp_i.wait()
    pltpu.sync_copy(rows_i, out_i_hbm.at[pl.ds(base, _B_PER_W)])


def kernel(real_table, imag_table, x):
    real_embed, imag_embed = _dual_gather(real_table, imag_table,
                                          x.astype(jnp.int32))
    return (real_embed, imag_embed)


# speedup vs baseline: 1.4973x; 1.4973x over previous
import functools

import jax
import jax.numpy as jnp
from jax import lax
from jax.experimental import pallas as pl
from jax.experimental.pallas import tpu as pltpu
from jax.experimental.pallas import tpu_sc as plsc

_VOCAB = 1000000
_FEATURES = 32
_BATCH = 16384

_info = plsc.get_sparse_core_info()
_NC, _NS = _info.num_cores, _info.num_subcores
_NW = _NC * _NS
_B_PER_W = _BATCH // _NW
_CHUNK = 256

_mesh = plsc.VectorSubcoreMesh(core_axis_name="c", subcore_axis_name="s")


@functools.partial(
    pl.kernel,
    mesh=_mesh,
    out_type=(
        jax.ShapeDtypeStruct((_BATCH, _FEATURES), jnp.float32),
        jax.ShapeDtypeStruct((_BATCH, _FEATURES), jnp.float32),
    ),
    scratch_types=[
        pltpu.VMEM((_B_PER_W,), jnp.int32),
        pltpu.VMEM((_CHUNK, _FEATURES), jnp.float32),
        pltpu.VMEM((_CHUNK, _FEATURES), jnp.float32),
        pltpu.SemaphoreType.DMA,
        pltpu.SemaphoreType.DMA,
    ],
)
def _dual_gather(real_hbm, imag_hbm, x_hbm, out_r_hbm, out_i_hbm,
                 idx_v, rows_r, rows_i, sem_r, sem_i):
    wid = lax.axis_index("s") * _NC + lax.axis_index("c")
    base = wid * _B_PER_W
    pltpu.sync_copy(x_hbm.at[pl.ds(base, _B_PER_W)], idx_v)

    for c in range(_B_PER_W // _CHUNK):
        def fire(g, carry):
            vec = idx_v[pl.ds(c * _CHUNK + g * 16, 16)]
            for u in range(16):
                s = vec[u]
                i = g * 16 + u
                pltpu.async_copy(real_hbm.at[s], rows_r.at[i], sem_r)
                pltpu.async_copy(imag_hbm.at[s], rows_i.at[i], sem_i)
            return carry

        lax.fori_loop(0, _CHUNK // 16, fire, 0)

        def drain(i, carry):
            pltpu.make_async_copy(real_hbm.at[0], rows_r.at[i], sem_r).wait()
            pltpu.make_async_copy(imag_hbm.at[0], rows_i.at[i], sem_i).wait()
            return carry

        lax.fori_loop(0, _CHUNK, drain, 0, unroll=4)
        pltpu.sync_copy(rows_r, out_r_hbm.at[pl.ds(base + c * _CHUNK, _CHUNK)])
        pltpu.sync_copy(rows_i, out_i_hbm.at[pl.ds(base + c * _CHUNK, _CHUNK)])


def kernel(real_table, imag_table, x):
    real_embed, imag_embed = _dual_gather(real_table, imag_table,
                                          x.astype(jnp.int32))
    return (real_embed, imag_embed)
